# Initial kernel scaffold; baseline (speedup 1.0000x reference)
import jax
import jax.numpy as jnp
from jax.experimental import pallas as pl


def kernel(h, x, edge_index, edge_attr, W1, b1, W2, b2, W3):
    raise NotImplementedError("write your pallas kernel here")


# speedup vs baseline: 2.6200x; 2.6200x over previous
import functools

import jax
import jax.numpy as jnp
from jax import lax
from jax.experimental import pallas as pl
from jax.experimental.pallas import tpu as pltpu
from jax.experimental.pallas import tpu_sc as plsc

N = 10000
E = 320000
H = 128
NC = 2
NS = 16
NW = NC * NS
EPW = E // NW
GCH = 80
ICH = 2000
F32 = jnp.float32

_mesh = plsc.VectorSubcoreMesh(core_axis_name="c", subcore_axis_name="s")
_sc_params = pltpu.CompilerParams(needs_layout_passes=False)


def _worker_id():
    return lax.axis_index("s") * NC + lax.axis_index("c")


def _precompute_body(h_ref, w1r_ref, w1c_ref, b1_ref, a_ref, b_ref):
    a_ref[...] = jnp.dot(h_ref[...], w1r_ref[...], preferred_element_type=F32)
    b_ref[...] = (jnp.dot(h_ref[...], w1c_ref[...], preferred_element_type=F32)
                  + b1_ref[...])


def _precompute(h, w1r, w1c, b1):
    return pl.pallas_call(
        _precompute_body,
        out_shape=[jax.ShapeDtypeStruct((N, H), F32),
                   jax.ShapeDtypeStruct((N, H), F32)],
    )(h, w1r, w1c, b1)


def _gather_body(ap_hbm, bp_hbm, row_hbm, col_hbm, x0_hbm, x1_hbm, x2_hbm,
                 ga_hbm, gb_hbm, d0_hbm, d1_hbm, d2_hbm, rad_hbm,
                 idxr, idxc, bufa, bufb, x0v, x1v, x2v,
                 d0v, d1v, d2v, radv, sema, semb):
    wid = _worker_id()
    base0 = wid * EPW
    pltpu.sync_copy(x0_hbm, x0v)
    pltpu.sync_copy(x1_hbm, x1v)
    pltpu.sync_copy(x2_hbm, x2v)

    def outer(ci, _):
        base = base0 + ci * ICH
        pltpu.sync_copy(row_hbm.at[pl.ds(base, ICH)], idxr)
        pltpu.sync_copy(col_hbm.at[pl.ds(base, ICH)], idxc)

        def inner(k, _):
            off = k * GCH
            ca = pltpu.async_copy(ap_hbm.at[idxr.at[pl.ds(off, GCH)]], bufa, sema)
            cb = pltpu.async_copy(bp_hbm.at[idxc.at[pl.ds(off, GCH)]], bufb, semb)
            ca.wait()
            cb.wait()
            pltpu.sync_copy(bufa, ga_hbm.at[pl.ds(base + off, GCH)])
            pltpu.sync_copy(bufb, gb_hbm.at[pl.ds(base + off, GCH)])
            return 0

        lax.fori_loop(0, ICH // GCH, inner, 0)

        def coords(j, _):
            sl = pl.ds(j * 16, 16)
            r = idxr[sl]
            c = idxc[sl]
            d0 = plsc.load_gather(x0v, [r]) - plsc.load_gather(x0v, [c])
            d1 = plsc.load_gather(x1v, [r]) - plsc.load_gather(x1v, [c])
            d2 = plsc.load_gather(x2v, [r]) - plsc.load_gather(x2v, [c])
            d0v[sl] = d0
            d1v[sl] = d1
            d2v[sl] = d2
            radv[sl] = d0 * d0 + d1 * d1 + d2 * d2
            return 0

        lax.fori_loop(0, ICH // 16, coords, 0)
        pltpu.sync_copy(d0v, d0_hbm.at[pl.ds(base, ICH)])
        pltpu.sync_copy(d1v, d1_hbm.at[pl.ds(base, ICH)])
        pltpu.sync_copy(d2v, d2_hbm.at[pl.ds(base, ICH)])
        pltpu.sync_copy(radv, rad_hbm.at[pl.ds(base, ICH)])
        return 0

    lax.fori_loop(0, EPW // ICH, outer, 0)


_gather = functools.partial(
    pl.kernel,
    out_type=[jax.ShapeDtypeStruct((E, H), F32),
              jax.ShapeDtypeStruct((E, H), F32),
              jax.ShapeDtypeStruct((E,), F32),
              jax.ShapeDtypeStruct((E,), F32),
              jax.ShapeDtypeStruct((E,), F32),
              jax.ShapeDtypeStruct((E,), F32)],
    mesh=_mesh,
    scratch_types=[
        pltpu.VMEM((ICH,), jnp.int32),
        pltpu.VMEM((ICH,), jnp.int32),
        pltpu.VMEM((GCH, H), F32),
        pltpu.VMEM((GCH, H), F32),
        pltpu.VMEM((N,), F32),
        pltpu.VMEM((N,), F32),
        pltpu.VMEM((N,), F32),
        pltpu.VMEM((ICH,), F32),
        pltpu.VMEM((ICH,), F32),
        pltpu.VMEM((ICH,), F32),
        pltpu.VMEM((ICH,), F32),
        pltpu.SemaphoreType.DMA,
        pltpu.SemaphoreType.DMA,
    ],
    compiler_params=_sc_params,
)(_gather_body)


EB = 2000


def _mlp_body(ga_ref, gb_ref, ea_ref, d0_ref, d1_ref, d2_ref, rad_ref,
              w1ea_ref, w2_ref, b2_ref, w3_ref,
              t0_ref, t1_ref, t2_ref):
    radial = rad_ref[...]
    u = (ga_ref[...] + gb_ref[...]
         + jnp.dot(ea_ref[...], w1ea_ref[...], preferred_element_type=F32)
         + radial * w1ea_ref[7:8, :])
    t = u * jax.nn.sigmoid(u)
    v = jnp.dot(t, w2_ref[...], preferred_element_type=F32) + b2_ref[...]
    t = v * jax.nn.sigmoid(v)
    s = jnp.sum(t * w3_ref[...], axis=1, keepdims=True)
    f = s / (jnp.sqrt(radial + 1e-8) + 1.0)
    t0_ref[...] = d0_ref[...] * f
    t1_ref[...] = d1_ref[...] * f
    t2_ref[...] = d2_ref[...] * f


def _mlp(ga, gb, ea, d0, d1, d2, rad, w1ea, w2, b2, w3):
    grid = (E // EB,)
    col_spec = pl.BlockSpec((EB, 1), lambda i: (i, 0))
    return pl.pallas_call(
        _mlp_body,
        grid=grid,
        in_specs=[
            pl.BlockSpec((EB, H), lambda i: (i, 0)),
            pl.BlockSpec((EB, H), lambda i: (i, 0)),
            pl.BlockSpec((EB, 8), lambda i: (i, 0)),
            col_spec, col_spec, col_spec, col_spec,
            pl.BlockSpec((8, H), lambda i: (0, 0)),
            pl.BlockSpec((H, H), lambda i: (0, 0)),
            pl.BlockSpec((1, H), lambda i: (0, 0)),
            pl.BlockSpec((1, H), lambda i: (0, 0)),
        ],
        out_specs=[col_spec, col_spec, col_spec],
        out_shape=[jax.ShapeDtypeStruct((E, 1), F32)] * 3,
    )(ga, gb, ea, d0, d1, d2, rad, w1ea, w2, b2, w3)


def _scatter_body(row_hbm, t0_hbm, t1_hbm, t2_hbm, parts_hbm,
                  rowv, v0, v1, v2, acc0, acc1, acc2):
    wid = _worker_id()
    base0 = wid * EPW

    def zero(i, _):
        sl = pl.ds(i * 16, 16)
        z = jnp.zeros((16,), F32)
        acc0[sl] = z
        acc1[sl] = z
        acc2[sl] = z
        return 0

    lax.fori_loop(0, N // 16, zero, 0)

    def outer(ci, _):
        base = base0 + ci * ICH
        pltpu.sync_copy(row_hbm.at[pl.ds(base, ICH)], rowv)
        pltpu.sync_copy(t0_hbm.at[pl.ds(base, ICH)], v0)
        pltpu.sync_copy(t1_hbm.at[pl.ds(base, ICH)], v1)
        pltpu.sync_copy(t2_hbm.at[pl.ds(base, ICH)], v2)

        def inner(j, _):
            sl = pl.ds(j * 16, 16)
            r = rowv[sl]
            plsc.addupdate_scatter(acc0, [r], v0[sl])
            plsc.addupdate_scatter(acc1, [r], v1[sl])
            plsc.addupdate_scatter(acc2, [r], v2[sl])
            return 0

        lax.fori_loop(0, ICH // 16, inner, 0)
        return 0

    lax.fori_loop(0, EPW // ICH, outer, 0)

    pbase = wid * (3 * N)
    pltpu.sync_copy(acc0, parts_hbm.at[pl.ds(pbase, N)])
    pltpu.sync_copy(acc1, parts_hbm.at[pl.ds(pbase + N, N)])
    pltpu.sync_copy(acc2, parts_hbm.at[pl.ds(pbase + 2 * N, N)])


_scatter = functools.partial(
    pl.kernel,
    out_type=jax.ShapeDtypeStruct((NW * 3 * N,), F32),
    mesh=_mesh,
    scratch_types=[
        pltpu.VMEM((ICH,), jnp.int32),
        pltpu.VMEM((ICH,), F32),
        pltpu.VMEM((ICH,), F32),
        pltpu.VMEM((ICH,), F32),
        pltpu.VMEM((N,), F32),
        pltpu.VMEM((N,), F32),
        pltpu.VMEM((N,), F32),
    ],
    compiler_params=_sc_params,
)(_scatter_body)


def _combine_body(parts_ref, xt_ref, out_ref):
    s = jnp.sum(parts_ref[...], axis=0)
    out_ref[...] = xt_ref[...] + s * 0.01


def _combine(parts, xt):
    return pl.pallas_call(
        _combine_body,
        out_shape=jax.ShapeDtypeStruct((3, N), F32),
    )(parts, xt)


def kernel(h, x, edge_index, edge_attr, W1, b1, W2, b2, W3):
    row = edge_index[0].astype(jnp.int32)
    col = edge_index[1].astype(jnp.int32)
    xt = x.T
    ea = jnp.pad(edge_attr, ((0, 0), (0, 1)))
    w1r = W1[:H]
    w1c = W1[H:2 * H]
    w1ea = W1[2 * H:]
    ap, bp = _precompute(h, w1r, w1c, b1.reshape(1, H))
    ga, gb, d0, d1, d2, rad = _gather(ap, bp, row, col, xt[0], xt[1], xt[2])
    t0, t1, t2 = _mlp(ga, gb, ea,
                      d0.reshape(E, 1), d1.reshape(E, 1), d2.reshape(E, 1),
                      rad.reshape(E, 1), w1ea, W2, b2.reshape(1, H),
                      W3.reshape(1, H))
    parts = _scatter(row, t0.reshape(E), t1.reshape(E), t2.reshape(E))
    xnt = _combine(parts.reshape(NW, 3, N), xt)
    return (h, xnt.T)


# speedup vs baseline: 4.5485x; 1.7361x over previous
import functools

import jax
import jax.numpy as jnp
from jax import lax
from jax.experimental import pallas as pl
from jax.experimental.pallas import tpu as pltpu
from jax.experimental.pallas import tpu_sc as plsc

N = 10000
E = 320000
H = 128
NC = 2
NS = 16
NW = NC * NS
EPW = E // NW
GCH = 80
ICH = 2000
F32 = jnp.float32

_mesh = plsc.VectorSubcoreMesh(core_axis_name="c", subcore_axis_name="s")
_sc_params = pltpu.CompilerParams(needs_layout_passes=False)


def _worker_id():
    return lax.axis_index("s") * NC + lax.axis_index("c")


def _precompute_body(h_ref, w1r_ref, w1c_ref, b1_ref, a_ref, b_ref):
    a_ref[...] = jnp.dot(h_ref[...], w1r_ref[...], preferred_element_type=F32)
    b_ref[...] = (jnp.dot(h_ref[...], w1c_ref[...], preferred_element_type=F32)
                  + b1_ref[...])


def _precompute(h, w1r, w1c, b1):
    return pl.pallas_call(
        _precompute_body,
        out_shape=[jax.ShapeDtypeStruct((N, H), F32),
                   jax.ShapeDtypeStruct((N, H), F32)],
    )(h, w1r, w1c, b1)


def _gather_body(ap_hbm, bp_hbm, row_hbm, col_hbm, x0_hbm, x1_hbm, x2_hbm,
                 ga_hbm, gb_hbm, d0_hbm, d1_hbm, d2_hbm, rad_hbm,
                 idxr, idxc, bufa, bufb, x0v, x1v, x2v,
                 d0v, d1v, d2v, radv, sema, semb):
    wid = _worker_id()
    base0 = wid * EPW
    pltpu.sync_copy(x0_hbm, x0v)
    pltpu.sync_copy(x1_hbm, x1v)
    pltpu.sync_copy(x2_hbm, x2v)

    def outer(ci, _):
        base = base0 + ci * ICH
        pltpu.sync_copy(row_hbm.at[pl.ds(base, ICH)], idxr)
        pltpu.sync_copy(col_hbm.at[pl.ds(base, ICH)], idxc)

        def inner(k, _):
            off = k * GCH
            ca = pltpu.async_copy(ap_hbm.at[idxr.at[pl.ds(off, GCH)]], bufa, sema)
            cb = pltpu.async_copy(bp_hbm.at[idxc.at[pl.ds(off, GCH)]], bufb, semb)
            ca.wait()
            cb.wait()
            pltpu.sync_copy(bufa, ga_hbm.at[pl.ds(base + off, GCH)])
            pltpu.sync_copy(bufb, gb_hbm.at[pl.ds(base + off, GCH)])
            return 0

        lax.fori_loop(0, ICH // GCH, inner, 0)

        def coords(j, _):
            sl = pl.ds(j * 16, 16)
            r = idxr[sl]
            c = idxc[sl]
            d0 = plsc.load_gather(x0v, [r]) - plsc.load_gather(x0v, [c])
            d1 = plsc.load_gather(x1v, [r]) - plsc.load_gather(x1v, [c])
            d2 = plsc.load_gather(x2v, [r]) - plsc.load_gather(x2v, [c])
            d0v[sl] = d0
            d1v[sl] = d1
            d2v[sl] = d2
            radv[sl] = d0 * d0 + d1 * d1 + d2 * d2
            return 0

        lax.fori_loop(0, ICH // 16, coords, 0)
        pltpu.sync_copy(d0v, d0_hbm.at[pl.ds(base, ICH)])
        pltpu.sync_copy(d1v, d1_hbm.at[pl.ds(base, ICH)])
        pltpu.sync_copy(d2v, d2_hbm.at[pl.ds(base, ICH)])
        pltpu.sync_copy(radv, rad_hbm.at[pl.ds(base, ICH)])
        return 0

    lax.fori_loop(0, EPW // ICH, outer, 0)


_gather = functools.partial(
    pl.kernel,
    out_type=[jax.ShapeDtypeStruct((E, H), F32),
              jax.ShapeDtypeStruct((E, H), F32),
              jax.ShapeDtypeStruct((E,), F32),
              jax.ShapeDtypeStruct((E,), F32),
              jax.ShapeDtypeStruct((E,), F32),
              jax.ShapeDtypeStruct((E,), F32)],
    mesh=_mesh,
    scratch_types=[
        pltpu.VMEM((ICH,), jnp.int32),
        pltpu.VMEM((ICH,), jnp.int32),
        pltpu.VMEM((GCH, H), F32),
        pltpu.VMEM((GCH, H), F32),
        pltpu.VMEM((N,), F32),
        pltpu.VMEM((N,), F32),
        pltpu.VMEM((N,), F32),
        pltpu.VMEM((ICH,), F32),
        pltpu.VMEM((ICH,), F32),
        pltpu.VMEM((ICH,), F32),
        pltpu.VMEM((ICH,), F32),
        pltpu.SemaphoreType.DMA,
        pltpu.SemaphoreType.DMA,
    ],
    compiler_params=_sc_params,
)(_gather_body)


EB = 512


def _mlp_body(ga_ref, gb_ref, ea_ref, d0_ref, d1_ref, d2_ref, rad_ref,
              w1ea_ref, w1rad_ref, w2_ref, b2_ref, w3_ref,
              t0_ref, t1_ref, t2_ref):
    radial = rad_ref[...].reshape(EB, 1)
    u = (ga_ref[...] + gb_ref[...]
         + jnp.dot(ea_ref[...], w1ea_ref[...], preferred_element_type=F32)
         + radial * w1rad_ref[...])
    t = u * jax.nn.sigmoid(u)
    v = jnp.dot(t, w2_ref[...], preferred_element_type=F32) + b2_ref[...]
    t = v * jax.nn.sigmoid(v)
    s = lax.dot_general(w3_ref[...], t, (((1,), (1,)), ((), ())),
                        preferred_element_type=F32).reshape(EB)
    rad1 = rad_ref[...]
    f = s / (jnp.sqrt(rad1 + 1e-8) + 1.0)
    t0_ref[...] = d0_ref[...] * f
    t1_ref[...] = d1_ref[...] * f
    t2_ref[...] = d2_ref[...] * f


def _mlp(ga, gb, ea, d0, d1, d2, rad, w1ea, w1rad, w2, b2, w3):
    grid = (E // EB,)
    vec_spec = pl.BlockSpec((EB,), lambda i: (i,))
    return pl.pallas_call(
        _mlp_body,
        grid=grid,
        in_specs=[
            pl.BlockSpec((EB, H), lambda i: (i, 0)),
            pl.BlockSpec((EB, H), lambda i: (i, 0)),
            pl.BlockSpec((EB, 7), lambda i: (i, 0)),
            vec_spec, vec_spec, vec_spec, vec_spec,
            pl.BlockSpec((7, H), lambda i: (0, 0)),
            pl.BlockSpec((1, H), lambda i: (0, 0)),
            pl.BlockSpec((H, H), lambda i: (0, 0)),
            pl.BlockSpec((1, H), lambda i: (0, 0)),
            pl.BlockSpec((1, H), lambda i: (0, 0)),
        ],
        out_specs=[vec_spec, vec_spec, vec_spec],
        out_shape=[jax.ShapeDtypeStruct((E,), F32)] * 3,
    )(ga, gb, ea, d0, d1, d2, rad, w1ea, w1rad, w2, b2, w3)


def _scatter_body(row_hbm, t0_hbm, t1_hbm, t2_hbm, parts_hbm,
                  rowv, v0, v1, v2, acc0, acc1, acc2):
    wid = _worker_id()
    base0 = wid * EPW

    def zero(i, _):
        sl = pl.ds(i * 16, 16)
        z = jnp.zeros((16,), F32)
        acc0[sl] = z
        acc1[sl] = z
        acc2[sl] = z
        return 0

    lax.fori_loop(0, N // 16, zero, 0)

    def outer(ci, _):
        base = base0 + ci * ICH
        pltpu.sync_copy(row_hbm.at[pl.ds(base, ICH)], rowv)
        pltpu.sync_copy(t0_hbm.at[pl.ds(base, ICH)], v0)
        pltpu.sync_copy(t1_hbm.at[pl.ds(base, ICH)], v1)
        pltpu.sync_copy(t2_hbm.at[pl.ds(base, ICH)], v2)

        def inner(j, _):
            sl = pl.ds(j * 16, 16)
            r = rowv[sl]
            plsc.addupdate_scatter(acc0, [r], v0[sl])
            plsc.addupdate_scatter(acc1, [r], v1[sl])
            plsc.addupdate_scatter(acc2, [r], v2[sl])
            return 0

        lax.fori_loop(0, ICH // 16, inner, 0)
        return 0

    lax.fori_loop(0, EPW // ICH, outer, 0)

    pbase = wid * (3 * N)
    pltpu.sync_copy(acc0, parts_hbm.at[pl.ds(pbase, N)])
    pltpu.sync_copy(acc1, parts_hbm.at[pl.ds(pbase + N, N)])
    pltpu.sync_copy(acc2, parts_hbm.at[pl.ds(pbase + 2 * N, N)])


_scatter = functools.partial(
    pl.kernel,
    out_type=jax.ShapeDtypeStruct((NW * 3 * N,), F32),
    mesh=_mesh,
    scratch_types=[
        pltpu.VMEM((ICH,), jnp.int32),
        pltpu.VMEM((ICH,), F32),
        pltpu.VMEM((ICH,), F32),
        pltpu.VMEM((ICH,), F32),
        pltpu.VMEM((N,), F32),
        pltpu.VMEM((N,), F32),
        pltpu.VMEM((N,), F32),
    ],
    compiler_params=_sc_params,
)(_scatter_body)


def _combine_body(parts_ref, xt_ref, out_ref):
    s = jnp.sum(parts_ref[...], axis=0)
    out_ref[...] = xt_ref[...] + s * 0.01


def _combine(parts, xt):
    return pl.pallas_call(
        _combine_body,
        out_shape=jax.ShapeDtypeStruct((3, N), F32),
    )(parts, xt)


def kernel(h, x, edge_index, edge_attr, W1, b1, W2, b2, W3):
    row = edge_index[0].astype(jnp.int32)
    col = edge_index[1].astype(jnp.int32)
    xt = x.T
    w1r = W1[:H]
    w1c = W1[H:2 * H]
    w1ea = W1[2 * H:2 * H + 7]
    w1rad = W1[2 * H + 7:]
    ap, bp = _precompute(h, w1r, w1c, b1.reshape(1, H))
    ga, gb, d0, d1, d2, rad = _gather(ap, bp, row, col, xt[0], xt[1], xt[2])
    t0, t1, t2 = _mlp(ga, gb, edge_attr, d0, d1, d2, rad,
                      w1ea, w1rad, W2, b2.reshape(1, H), W3.reshape(1, H))
    parts = _scatter(row, t0, t1, t2)
    xnt = _combine(parts.reshape(NW, 3, N), xt)
    return (h, xnt.T)
